# baseline (speedup 1.0000x reference)
import functools

import jax
import jax.numpy as jnp
from jax import lax
from jax.experimental import pallas as pl
from jax.experimental.pallas import tpu as pltpu
from jax.experimental.pallas import tpu_sc as plsc

_N = 10000
_NP = 10240
_E = 320000
_H = 128
_G = 64

_K = 80
_NW = 32
_RW = _E // (_NW * _K)
_CS = _RW + 1
_CP = _CS + 1
_CR = 128
_EW = _CR * _K
_TNP = _NP // 16
_WB = 80

_sc_mesh = plsc.VectorSubcoreMesh(core_axis_name="c", subcore_axis_name="s")


@functools.partial(
    pl.kernel,
    out_type=jax.ShapeDtypeStruct((2, _NP, _H), jnp.float32),
    mesh=_sc_mesh,
    scratch_types=[
        pltpu.VMEM((_EW,), jnp.int32),
        pltpu.VMEM((_CR, _K), jnp.int32),
        pltpu.VMEM((_K, _H), jnp.float32),
        pltpu.VMEM((_K, _H), jnp.float32),
        pltpu.VMEM_SHARED((_NP, _H), jnp.float32),
        pltpu.SemaphoreType.DMA,
        pltpu.SemaphoreType.DMA,
    ],
)
def _agg(y, srcflat, dst3d, zeros_hbm, out, sidx, didx, buf0, buf1, acc,
         sem0, sem1):
    c = lax.axis_index("c")
    s = lax.axis_index("s")
    wid = s * 2 + c
    pltpu.sync_copy(zeros_hbm, buf0)
    for i in range(8):
        pltpu.sync_copy(buf0, acc.at[pl.ds(s * _TNP + i * _WB, _WB)])
    pltpu.sync_copy(srcflat.at[pl.ds(wid * _EW, _EW)], sidx)
    pltpu.sync_copy(dst3d.at[wid], didx)
    plsc.subcore_barrier()

    pltpu.async_copy(y.at[sidx.at[pl.ds(0, _K)]], buf0, sem0)

    def body(j, carry):
        b = 2 * j
        pltpu.async_copy(y.at[sidx.at[pl.ds((b + 1) * _K, _K)]], buf1, sem1)
        pltpu.make_async_copy(y.at[sidx.at[pl.ds(b * _K, _K)]], buf0,
                              sem0).wait()
        pltpu.sync_copy(buf0, acc.at[didx.at[b]], add=True)
        pltpu.async_copy(y.at[sidx.at[pl.ds((b + 2) * _K, _K)]], buf0, sem0)
        pltpu.make_async_copy(y.at[sidx.at[pl.ds((b + 1) * _K, _K)]], buf1,
                              sem1).wait()
        pltpu.sync_copy(buf1, acc.at[didx.at[b + 1]], add=True)
        return carry

    lax.fori_loop(0, _CS // 2, body, 0)
    pltpu.make_async_copy(y.at[sidx.at[pl.ds(_CS * _K, _K)]], buf0,
                          sem0).wait()
    plsc.subcore_barrier()
    for i in range(8):
        pltpu.sync_copy(acc.at[pl.ds(s * _TNP + i * _WB, _WB)], buf0)
        pltpu.sync_copy(buf0, out.at[c, pl.ds(s * _TNP + i * _WB, _WB)])


@functools.partial(
    pl.kernel,
    out_type=jax.ShapeDtypeStruct((2, _NP, _H), jnp.float32),
    mesh=_sc_mesh,
    scratch_types=[
        pltpu.VMEM((_RW, _K), jnp.int32),
        pltpu.VMEM((_K, _H), jnp.float32),
        pltpu.VMEM_SHARED((_NP, _H), jnp.float32),
    ],
)
def _deg(dst3d, zeros_hbm, ones_hbm, out, didx, buf, acc):
    c = lax.axis_index("c")
    s = lax.axis_index("s")
    wid = s * 2 + c
    pltpu.sync_copy(zeros_hbm, buf)
    for i in range(8):
        pltpu.sync_copy(buf, acc.at[pl.ds(s * _TNP + i * _WB, _WB)])
    pltpu.sync_copy(dst3d.at[wid], didx)
    pltpu.sync_copy(ones_hbm, buf)
    plsc.subcore_barrier()

    def body(j, carry):
        pltpu.sync_copy(buf, acc.at[didx.at[j]], add=True)
        return carry

    lax.fori_loop(0, _RW, body, 0)
    plsc.subcore_barrier()
    for i in range(8):
        pltpu.sync_copy(acc.at[pl.ds(s * _TNP + i * _WB, _WB)], buf)
        pltpu.sync_copy(buf, out.at[c, pl.ds(s * _TNP + i * _WB, _WB)])


def _tc_scale(x_ref, w_ref, degp_ref, y_ref, dinv_ref):
    deg = degp_ref[0, : _N] + degp_ref[1, : _N] + 1.0
    dinv = lax.rsqrt(jnp.maximum(deg, 1.0))
    xw = jnp.dot(x_ref[...], w_ref[...], preferred_element_type=jnp.float32)
    y_ref[...] = xw * dinv
    dinv_ref[...] = dinv


def _tc_mid(accp_ref, y_ref, dinv_ref, b_ref, w_ref, y2_ref):
    acc = accp_ref[0, : _N] + accp_ref[1, : _N] + y_ref[...]
    h = jnp.maximum(acc * dinv_ref[...] + b_ref[...], 0.0)
    xw = jnp.dot(h, w_ref[...], preferred_element_type=jnp.float32)
    y2_ref[...] = xw * dinv_ref[...]


def _tc_final(accp_ref, y2_ref, dinv_ref, b_ref, batch_ref,
              wl1_ref, bl1_ref, wl2_ref, bl2_ref, out_ref):
    acc = accp_ref[0, : _N] + accp_ref[1, : _N] + y2_ref[...]
    h = jnp.maximum(acc * dinv_ref[...] + b_ref[...], 0.0)
    seg = lax.broadcasted_iota(jnp.int32, (_G, _N), 0)
    p = (batch_ref[...] == seg).astype(jnp.float32)
    g = jnp.dot(p, h, preferred_element_type=jnp.float32)
    g1 = jnp.maximum(
        jnp.dot(g, wl1_ref[...], preferred_element_type=jnp.float32)
        + bl1_ref[...], 0.0)
    out_ref[...] = (
        jnp.dot(g1, wl2_ref[...], preferred_element_type=jnp.float32)
        + bl2_ref[...])


def kernel(x, edge_index, batch, Wc1, bc1, Wc2, bc2, Wl1, bl1, Wl2, bl2):
    dst3d = edge_index[1].reshape(_NW, _RW, _K)
    npad = _CR - _RW
    srcflat = jnp.concatenate(
        [edge_index[0].reshape(_NW, _RW, _K),
         jnp.zeros((_NW, npad, _K), jnp.int32)], 1).reshape(_NW * _EW)
    dst3a = jnp.concatenate(
        [dst3d, jnp.full((_NW, npad, _K), _N, jnp.int32)], 1)
    ones_h = jnp.ones((_K, _H), jnp.float32)
    zeros_h = jnp.zeros((_WB, _H), jnp.float32)

    degp = _deg(dst3d, zeros_h, ones_h)

    y1, dinv = pl.pallas_call(
        _tc_scale,
        out_shape=(
            jax.ShapeDtypeStruct((_N, _H), jnp.float32),
            jax.ShapeDtypeStruct((_N, _H), jnp.float32),
        ),
    )(x, Wc1, degp)

    accp1 = _agg(y1, srcflat, dst3a, zeros_h)

    y2 = pl.pallas_call(
        _tc_mid,
        out_shape=jax.ShapeDtypeStruct((_N, _H), jnp.float32),
    )(accp1, y1, dinv, bc1.reshape(1, _H), Wc2)

    accp2 = _agg(y2, srcflat, dst3a, zeros_h)

    out = pl.pallas_call(
        _tc_final,
        out_shape=jax.ShapeDtypeStruct((_G, 10), jnp.float32),
    )(accp2, y2, dinv, bc2.reshape(1, _H), batch.reshape(1, _N),
      Wl1, bl1.reshape(1, _H), Wl2, bl2.reshape(1, 10))

    return out


# speedup vs baseline: 1.1608x; 1.1608x over previous
import functools

import jax
import jax.numpy as jnp
from jax import lax
from jax.experimental import pallas as pl
from jax.experimental.pallas import tpu as pltpu
from jax.experimental.pallas import tpu_sc as plsc

_N = 10000
_NP = 10240
_E = 320000
_H = 128
_G = 64

_K = 80
_NW = 32
_RW = _E // (_NW * _K)
_CS = _RW + 1
_CP = _CS + 1
_CR = 128
_EW = _CR * _K
_TNP = _NP // 16
_WB = 80

_sc_mesh = plsc.VectorSubcoreMesh(core_axis_name="c", subcore_axis_name="s")


@functools.partial(
    pl.kernel,
    out_type=jax.ShapeDtypeStruct((2, _NP, _H), jnp.float32),
    mesh=_sc_mesh,
    scratch_types=[
        pltpu.VMEM((_EW,), jnp.int32),
        pltpu.VMEM((_CR, _K), jnp.int32),
        pltpu.VMEM((_K, _H), jnp.float32),
        pltpu.VMEM((_K, _H), jnp.float32),
        pltpu.VMEM_SHARED((_NP, _H), jnp.float32),
        pltpu.SemaphoreType.DMA,
        pltpu.SemaphoreType.DMA,
    ],
)
def _agg(y, srcflat, dst3d, zeros_hbm, out, sidx, didx, buf0, buf1, acc,
         sem0, sem1):
    c = lax.axis_index("c")
    s = lax.axis_index("s")
    wid = s * 2 + c
    pltpu.sync_copy(zeros_hbm, buf0)
    for i in range(8):
        pltpu.sync_copy(buf0, acc.at[pl.ds(s * _TNP + i * _WB, _WB)])
    pltpu.sync_copy(srcflat.at[pl.ds(wid * _EW, _EW)], sidx)
    pltpu.sync_copy(dst3d.at[wid], didx)
    plsc.subcore_barrier()

    def body(j, carry):
        b = 2 * j
        d0 = pltpu.async_copy(y.at[sidx.at[pl.ds(b * _K, _K)]], buf0, sem0)
        d1 = pltpu.async_copy(y.at[sidx.at[pl.ds((b + 1) * _K, _K)]], buf1,
                              sem1)
        d0.wait()
        d1.wait()
        pltpu.sync_copy(buf0, acc.at[didx.at[b]], add=True)
        pltpu.sync_copy(buf1, acc.at[didx.at[b + 1]], add=True)
        return carry

    lax.fori_loop(0, _CS // 2, body, 0)
    plsc.subcore_barrier()
    for i in range(8):
        pltpu.sync_copy(acc.at[pl.ds(s * _TNP + i * _WB, _WB)], buf0)
        pltpu.sync_copy(buf0, out.at[c, pl.ds(s * _TNP + i * _WB, _WB)])


@functools.partial(
    pl.kernel,
    out_type=jax.ShapeDtypeStruct((2, _NP, _H), jnp.float32),
    mesh=_sc_mesh,
    scratch_types=[
        pltpu.VMEM((_RW, _K), jnp.int32),
        pltpu.VMEM((_K, _H), jnp.float32),
        pltpu.VMEM_SHARED((_NP, _H), jnp.float32),
    ],
)
def _deg(dst3d, zeros_hbm, ones_hbm, out, didx, buf, acc):
    c = lax.axis_index("c")
    s = lax.axis_index("s")
    wid = s * 2 + c
    pltpu.sync_copy(zeros_hbm, buf)
    for i in range(8):
        pltpu.sync_copy(buf, acc.at[pl.ds(s * _TNP + i * _WB, _WB)])
    pltpu.sync_copy(dst3d.at[wid], didx)
    pltpu.sync_copy(ones_hbm, buf)
    plsc.subcore_barrier()

    def body(j, carry):
        pltpu.sync_copy(buf, acc.at[didx.at[j]], add=True)
        return carry

    lax.fori_loop(0, _RW, body, 0)
    plsc.subcore_barrier()
    for i in range(8):
        pltpu.sync_copy(acc.at[pl.ds(s * _TNP + i * _WB, _WB)], buf)
        pltpu.sync_copy(buf, out.at[c, pl.ds(s * _TNP + i * _WB, _WB)])


def _tc_mm(x_ref, w_ref, xw_ref):
    xw_ref[...] = jnp.dot(x_ref[...], w_ref[...],
                          preferred_element_type=jnp.float32)


def _tc_scale(xw_ref, degp_ref, y_ref, dinv_ref):
    deg = degp_ref[0, : _N] + degp_ref[1, : _N] + 1.0
    dinv = lax.rsqrt(jnp.maximum(deg, 1.0))
    y_ref[...] = xw_ref[...] * dinv
    dinv_ref[...] = dinv


def _tc_mid(accp_ref, y_ref, dinv_ref, b_ref, w_ref, y2_ref):
    acc = accp_ref[0, : _N] + accp_ref[1, : _N] + y_ref[...]
    h = jnp.maximum(acc * dinv_ref[...] + b_ref[...], 0.0)
    xw = jnp.dot(h, w_ref[...], preferred_element_type=jnp.float32)
    y2_ref[...] = xw * dinv_ref[...]


def _tc_final(accp_ref, y2_ref, dinv_ref, b_ref, batch_ref,
              wl1_ref, bl1_ref, wl2_ref, bl2_ref, out_ref):
    acc = accp_ref[0, : _N] + accp_ref[1, : _N] + y2_ref[...]
    h = jnp.maximum(acc * dinv_ref[...] + b_ref[...], 0.0)
    seg = lax.broadcasted_iota(jnp.int32, (_G, _N), 0)
    p = (batch_ref[...] == seg).astype(jnp.float32)
    g = jnp.dot(p, h, preferred_element_type=jnp.float32)
    g1 = jnp.maximum(
        jnp.dot(g, wl1_ref[...], preferred_element_type=jnp.float32)
        + bl1_ref[...], 0.0)
    out_ref[...] = (
        jnp.dot(g1, wl2_ref[...], preferred_element_type=jnp.float32)
        + bl2_ref[...])


def kernel(x, edge_index, batch, Wc1, bc1, Wc2, bc2, Wl1, bl1, Wl2, bl2):
    dst3d = edge_index[1].reshape(_NW, _RW, _K)
    npad = _CR - _RW
    srcflat = jnp.concatenate(
        [edge_index[0].reshape(_NW, _RW, _K),
         jnp.zeros((_NW, npad, _K), jnp.int32)], 1).reshape(_NW * _EW)
    padrows = _N + jnp.arange(_K, dtype=jnp.int32)
    dst3a = jnp.concatenate(
        [dst3d, jnp.broadcast_to(padrows, (_NW, npad, _K))], 1)
    ones_h = jnp.ones((_K, _H), jnp.float32)
    zeros_h = jnp.zeros((_WB, _H), jnp.float32)

    degp = _deg(dst3d, zeros_h, ones_h)
    xw1 = pl.pallas_call(
        _tc_mm,
        out_shape=jax.ShapeDtypeStruct((_N, _H), jnp.float32),
    )(x, Wc1)

    y1, dinv = pl.pallas_call(
        _tc_scale,
        out_shape=(
            jax.ShapeDtypeStruct((_N, _H), jnp.float32),
            jax.ShapeDtypeStruct((_N, _H), jnp.float32),
        ),
    )(xw1, degp)

    accp1 = _agg(y1, srcflat, dst3a, zeros_h)

    y2 = pl.pallas_call(
        _tc_mid,
        out_shape=jax.ShapeDtypeStruct((_N, _H), jnp.float32),
    )(accp1, y1, dinv, bc1.reshape(1, _H), Wc2)

    accp2 = _agg(y2, srcflat, dst3a, zeros_h)

    out = pl.pallas_call(
        _tc_final,
        out_shape=jax.ShapeDtypeStruct((_G, 10), jnp.float32),
    )(accp2, y2, dinv, bc2.reshape(1, _H), batch.reshape(1, _N),
      Wl1, bl1.reshape(1, _H), Wl2, bl2.reshape(1, 10))

    return out


# speedup vs baseline: 1.3651x; 1.1759x over previous
import functools

import jax
import jax.numpy as jnp
from jax import lax
from jax.experimental import pallas as pl
from jax.experimental.pallas import tpu as pltpu
from jax.experimental.pallas import tpu_sc as plsc

_N = 10000
_NP = 10240
_E = 320000
_H = 128
_G = 64

_K = 80
_NW = 32
_RW = _E // (_NW * _K)
_CS = _RW + 1
_CP = _CS + 1
_CR = 128
_EW = _CR * _K
_TNP = _NP // 16
_WB = 80

_sc_mesh = plsc.VectorSubcoreMesh(core_axis_name="c", subcore_axis_name="s")


@functools.partial(
    pl.kernel,
    out_type=jax.ShapeDtypeStruct((2, _NP, _H), jnp.float32),
    mesh=_sc_mesh,
    scratch_types=[
        pltpu.VMEM((_EW,), jnp.int32),
        pltpu.VMEM((_CR, _K), jnp.int32),
        pltpu.VMEM((_K, _H), jnp.float32),
        pltpu.VMEM((_K, _H), jnp.float32),
        pltpu.VMEM_SHARED((_NP, _H), jnp.float32),
        pltpu.SemaphoreType.DMA,
        pltpu.SemaphoreType.DMA,
    ],
)
def _agg(y, srcflat, dst3d, zeros_hbm, out, sidx, didx, buf0, buf1, acc,
         sem0, sem1):
    c = lax.axis_index("c")
    s = lax.axis_index("s")
    wid = s * 2 + c
    pltpu.sync_copy(zeros_hbm, buf0)
    for i in range(8):
        pltpu.sync_copy(buf0, acc.at[pl.ds(s * _TNP + i * _WB, _WB)])
    pltpu.sync_copy(srcflat.at[pl.ds(wid * _EW, _EW)], sidx)
    pltpu.sync_copy(dst3d.at[wid], didx)
    plsc.subcore_barrier()

    def body(j, carry):
        pltpu.async_copy(y.at[sidx.at[pl.ds(j * _K, _K)]], buf0, sem0).wait()
        pltpu.sync_copy(buf0, acc.at[didx.at[j]], add=True)
        return carry

    lax.fori_loop(0, _RW, body, 0)
    plsc.subcore_barrier()
    for i in range(8):
        pltpu.sync_copy(acc.at[pl.ds(s * _TNP + i * _WB, _WB)], buf0)
        pltpu.sync_copy(buf0, out.at[c, pl.ds(s * _TNP + i * _WB, _WB)])


@functools.partial(
    pl.kernel,
    out_type=jax.ShapeDtypeStruct((2, _NP, _H), jnp.float32),
    mesh=_sc_mesh,
    scratch_types=[
        pltpu.VMEM((_RW, _K), jnp.int32),
        pltpu.VMEM((_K, _H), jnp.float32),
        pltpu.VMEM_SHARED((_NP, _H), jnp.float32),
    ],
)
def _deg(dst3d, zeros_hbm, ones_hbm, out, didx, buf, acc):
    c = lax.axis_index("c")
    s = lax.axis_index("s")
    wid = s * 2 + c
    pltpu.sync_copy(zeros_hbm, buf)
    for i in range(8):
        pltpu.sync_copy(buf, acc.at[pl.ds(s * _TNP + i * _WB, _WB)])
    pltpu.sync_copy(dst3d.at[wid], didx)
    pltpu.sync_copy(ones_hbm, buf)
    plsc.subcore_barrier()

    def body(j, carry):
        pltpu.sync_copy(buf, acc.at[didx.at[j]], add=True)
        return carry

    lax.fori_loop(0, _RW, body, 0)
    plsc.subcore_barrier()
    for i in range(8):
        pltpu.sync_copy(acc.at[pl.ds(s * _TNP + i * _WB, _WB)], buf)
        pltpu.sync_copy(buf, out.at[c, pl.ds(s * _TNP + i * _WB, _WB)])


def _tc_mm(x_ref, w_ref, xw_ref):
    xw_ref[...] = jnp.dot(x_ref[...], w_ref[...],
                          preferred_element_type=jnp.float32)


def _tc_scale(xw_ref, degp_ref, y_ref, dinv_ref):
    deg = degp_ref[0, : _N] + degp_ref[1, : _N] + 1.0
    dinv = lax.rsqrt(jnp.maximum(deg, 1.0))
    y_ref[...] = xw_ref[...] * dinv
    dinv_ref[...] = dinv


def _tc_mid(accp_ref, y_ref, dinv_ref, b_ref, w_ref, y2_ref):
    acc = accp_ref[0, : _N] + accp_ref[1, : _N] + y_ref[...]
    h = jnp.maximum(acc * dinv_ref[...] + b_ref[...], 0.0)
    xw = jnp.dot(h, w_ref[...], preferred_element_type=jnp.float32)
    y2_ref[...] = xw * dinv_ref[...]


def _tc_final(accp_ref, y2_ref, dinv_ref, b_ref, batch_ref,
              wl1_ref, bl1_ref, wl2_ref, bl2_ref, out_ref):
    acc = accp_ref[0, : _N] + accp_ref[1, : _N] + y2_ref[...]
    h = jnp.maximum(acc * dinv_ref[...] + b_ref[...], 0.0)
    seg = lax.broadcasted_iota(jnp.int32, (_G, _N), 0)
    p = (batch_ref[...] == seg).astype(jnp.float32)
    g = jnp.dot(p, h, preferred_element_type=jnp.float32)
    g1 = jnp.maximum(
        jnp.dot(g, wl1_ref[...], preferred_element_type=jnp.float32)
        + bl1_ref[...], 0.0)
    out_ref[...] = (
        jnp.dot(g1, wl2_ref[...], preferred_element_type=jnp.float32)
        + bl2_ref[...])


def kernel(x, edge_index, batch, Wc1, bc1, Wc2, bc2, Wl1, bl1, Wl2, bl2):
    dst3d = edge_index[1].reshape(_NW, _RW, _K)
    npad = _CR - _RW
    srcflat = jnp.concatenate(
        [edge_index[0].reshape(_NW, _RW, _K),
         jnp.zeros((_NW, npad, _K), jnp.int32)], 1).reshape(_NW * _EW)
    padrows = _N + jnp.arange(_K, dtype=jnp.int32)
    dst3a = jnp.concatenate(
        [dst3d, jnp.broadcast_to(padrows, (_NW, npad, _K))], 1)
    ones_h = jnp.ones((_K, _H), jnp.float32)
    zeros_h = jnp.zeros((_WB, _H), jnp.float32)

    degp = _deg(dst3d, zeros_h, ones_h)
    xw1 = pl.pallas_call(
        _tc_mm,
        out_shape=jax.ShapeDtypeStruct((_N, _H), jnp.float32),
    )(x, Wc1)

    y1, dinv = pl.pallas_call(
        _tc_scale,
        out_shape=(
            jax.ShapeDtypeStruct((_N, _H), jnp.float32),
            jax.ShapeDtypeStruct((_N, _H), jnp.float32),
        ),
    )(xw1, degp)

    accp1 = _agg(y1, srcflat, dst3a, zeros_h)

    y2 = pl.pallas_call(
        _tc_mid,
        out_shape=jax.ShapeDtypeStruct((_N, _H), jnp.float32),
    )(accp1, y1, dinv, bc1.reshape(1, _H), Wc2)

    accp2 = _agg(y2, srcflat, dst3a, zeros_h)

    out = pl.pallas_call(
        _tc_final,
        out_shape=jax.ShapeDtypeStruct((_G, 10), jnp.float32),
    )(accp2, y2, dinv, bc2.reshape(1, _H), batch.reshape(1, _N),
      Wl1, bl1.reshape(1, _H), Wl2, bl2.reshape(1, 10))

    return out
